# baseline (speedup 1.0000x reference)
import functools
import jax
import jax.numpy as jnp
from jax import lax
from jax.experimental import pallas as pl
from jax.experimental.pallas import tpu as pltpu
from jax.experimental.pallas import tpu_sc as plsc

_NUM_SETS = 26
_SET_LEN = 10
_EMBED_DIM = 64
_BATCH = 16384
_NUM_ROWS = _NUM_SETS * _SET_LEN

_L = 16
_NC, _NS = 2, 16
_NW = _NC * _NS
_CT = _EMBED_DIM // 8
_BCH = 8
_TPU_B = _BATCH // (_BCH * 128)
_UNITS = _NUM_SETS * _CT * _BCH
_U_PER_W = _UNITS // _NW

_mesh = plsc.VectorSubcoreMesh(core_axis_name="c", subcore_axis_name="s")


@functools.partial(
    pl.kernel,
    out_type=jax.ShapeDtypeStruct((_NUM_SETS, _EMBED_DIM, _BATCH), jnp.float32),
    mesh=_mesh,
    compiler_params=pltpu.CompilerParams(
        needs_layout_passes=False, use_tc_tiling_on_sc=True
    ),
    scratch_types=[
        pltpu.VMEM((_NUM_ROWS * _EMBED_DIM,), jnp.float32),
        pltpu.VMEM((2, _BATCH), jnp.int32),
        pltpu.VMEM((1, 8, _TPU_B * 128), jnp.float32),
        pltpu.VMEM((1, 8, _TPU_B * 128), jnp.float32),
        pltpu.VMEM((1, 8, _TPU_B * 128), jnp.float32),
        pltpu.VMEM((1, 8, _TPU_B * 128), jnp.float32),
        pltpu.SemaphoreType.DMA,
        pltpu.SemaphoreType.DMA,
        pltpu.SemaphoreType.DMA,
        pltpu.SemaphoreType.DMA,
    ],
)
def _sc_gather(idx_hbm, table_hbm, out_hbm, table_v, idx_v, stg_a, stg_b,
               stg_c, stg_d, sem_a, sem_b, sem_c, sem_d):
    wid = lax.axis_index("s") * _NC + lax.axis_index("c")
    ubase = wid * _U_PER_W
    s0 = ubase // (_CT * _BCH)

    pltpu.sync_copy(table_hbm, table_v)
    s_last = (ubase + _U_PER_W - 1) // (_CT * _BCH)
    pltpu.sync_copy(idx_hbm.at[pl.ds(s0, 1)], idx_v.at[pl.ds(0, 1)])
    pltpu.sync_copy(idx_hbm.at[pl.ds(s_last, 1)], idx_v.at[pl.ds(1, 1)])

    lane = lax.iota(jnp.int32, _L)
    lane_c = jnp.minimum(lane, _SET_LEN - 1)

    def out_unit(s, ct, ch):
        return out_hbm.at[
            pl.ds(s, 1), pl.ds(ct * 8, 8), pl.ds(ch * (_TPU_B * 128), _TPU_B * 128)
        ]

    def do_unit(ul, stg, sem, drain):
        u = ubase + ul
        s = u // (_CT * _BCH)
        r = lax.rem(u, _CT * _BCH)
        ct = r // _BCH
        ch = lax.rem(r, _BCH)
        irow = jnp.where(s == s0, 0, 1)
        if drain:
            pltpu.make_async_copy(stg, out_unit(s, ct, ch), sem).wait()
        tvecs = [
            plsc.load_gather(
                table_v,
                [s * (_SET_LEN * _EMBED_DIM) + lane_c * _EMBED_DIM
                 + (ct * 8 + i)],
            )
            for i in range(8)
        ]

        @pl.loop(0, _TPU_B)
        def _(t):
            b0 = ch * (_TPU_B * 128) + t * 128
            ivs = [idx_v[irow, pl.ds(b0 + g * _L, _L)] for g in range(8)]
            for i in range(8):
                for g in range(8):
                    stg[0, i, pl.ds(t * 128 + g * _L, _L)] = jnp.take_along_axis(
                        tvecs[i], ivs[g], axis=0, mode="promise_in_bounds"
                    )

        pltpu.async_copy(stg, out_unit(s, ct, ch), sem)

    do_unit(0, stg_a, sem_a, False)
    do_unit(1, stg_b, sem_b, False)
    do_unit(2, stg_c, sem_c, False)
    do_unit(3, stg_d, sem_d, False)

    @pl.loop(4, _U_PER_W, step=4)
    def _(ul):
        do_unit(ul, stg_a, sem_a, True)
        do_unit(ul + 1, stg_b, sem_b, True)
        do_unit(ul + 2, stg_c, sem_c, True)
        do_unit(ul + 3, stg_d, sem_d, True)

    pltpu.make_async_copy(stg_a, out_unit(0, 0, 0), sem_a).wait()
    pltpu.make_async_copy(stg_b, out_unit(0, 0, 0), sem_b).wait()
    pltpu.make_async_copy(stg_c, out_unit(0, 0, 0), sem_c).wait()
    pltpu.make_async_copy(stg_d, out_unit(0, 0, 0), sem_d).wait()


def kernel(indices, table):
    idx_t = indices.T
    table_flat = table.reshape(-1)
    out = _sc_gather(idx_t, table_flat)
    return out.transpose(2, 0, 1)


# speedup vs baseline: 1.0778x; 1.0778x over previous
import functools
import jax
import jax.numpy as jnp
from jax import lax
from jax.experimental import pallas as pl
from jax.experimental.pallas import tpu as pltpu
from jax.experimental.pallas import tpu_sc as plsc

_NUM_SETS = 26
_SET_LEN = 10
_EMBED_DIM = 64
_BATCH = 16384
_NUM_ROWS = _NUM_SETS * _SET_LEN

_L = 16
_NC, _NS = 2, 16
_NW = _NC * _NS
_CT = _EMBED_DIM // 8
_BCH = 4
_TPU_B = _BATCH // (_BCH * 128)
_UNITS = _NUM_SETS * _CT * _BCH
_U_PER_W = _UNITS // _NW

_mesh = plsc.VectorSubcoreMesh(core_axis_name="c", subcore_axis_name="s")


@functools.partial(
    pl.kernel,
    out_type=jax.ShapeDtypeStruct((_NUM_SETS, _EMBED_DIM, _BATCH), jnp.float32),
    mesh=_mesh,
    compiler_params=pltpu.CompilerParams(
        needs_layout_passes=False, use_tc_tiling_on_sc=True
    ),
    scratch_types=[
        pltpu.VMEM((_NUM_ROWS * _EMBED_DIM,), jnp.float32),
        pltpu.VMEM((2, _BATCH), jnp.int32),
        pltpu.VMEM((1, 8, _TPU_B * 128), jnp.float32),
        pltpu.VMEM((1, 8, _TPU_B * 128), jnp.float32),
        pltpu.SemaphoreType.DMA,
        pltpu.SemaphoreType.DMA,
    ],
)
def _sc_gather(idx_hbm, table_hbm, out_hbm, table_v, idx_v, stg_a, stg_b,
               sem_a, sem_b):
    wid = lax.axis_index("s") * _NC + lax.axis_index("c")
    ubase = wid * _U_PER_W
    s0 = ubase // (_CT * _BCH)

    s_last = (ubase + _U_PER_W - 1) // (_CT * _BCH)
    tbl_cp = pltpu.async_copy(table_hbm, table_v, sem_a)
    i0_cp = pltpu.async_copy(idx_hbm.at[pl.ds(s0, 1)], idx_v.at[pl.ds(0, 1)],
                             sem_b)
    i1_cp = pltpu.async_copy(idx_hbm.at[pl.ds(s_last, 1)],
                             idx_v.at[pl.ds(1, 1)], sem_b)
    tbl_cp.wait()
    i0_cp.wait()
    i1_cp.wait()

    lane = lax.iota(jnp.int32, _L)
    lane_c = jnp.minimum(lane, _SET_LEN - 1)

    def out_unit(s, ct, ch):
        return out_hbm.at[
            pl.ds(s, 1), pl.ds(ct * 8, 8), pl.ds(ch * (_TPU_B * 128), _TPU_B * 128)
        ]

    def do_unit(ul, stg, sem, drain):
        u = ubase + ul
        s = u // (_CT * _BCH)
        r = lax.rem(u, _CT * _BCH)
        ct = r // _BCH
        ch = lax.rem(r, _BCH)
        irow = jnp.where(s == s0, 0, 1)
        if drain:
            pltpu.make_async_copy(stg, out_unit(s, ct, ch), sem).wait()
        tvecs = [
            plsc.load_gather(
                table_v,
                [s * (_SET_LEN * _EMBED_DIM) + lane_c * _EMBED_DIM
                 + (ct * 8 + i)],
            )
            for i in range(8)
        ]

        @pl.loop(0, _TPU_B)
        def _(t):
            b0 = ch * (_TPU_B * 128) + t * 128
            ivs = [idx_v[irow, pl.ds(b0 + g * _L, _L)] for g in range(8)]
            for i in range(8):
                for g in range(8):
                    stg[0, i, pl.ds(t * 128 + g * _L, _L)] = jnp.take_along_axis(
                        tvecs[i], ivs[g], axis=0, mode="promise_in_bounds"
                    )

        pltpu.async_copy(stg, out_unit(s, ct, ch), sem)

    do_unit(0, stg_a, sem_a, False)
    do_unit(1, stg_b, sem_b, False)

    @pl.loop(2, _U_PER_W, step=2)
    def _(ul):
        do_unit(ul, stg_a, sem_a, True)
        do_unit(ul + 1, stg_b, sem_b, True)

    pltpu.make_async_copy(stg_a, out_unit(0, 0, 0), sem_a).wait()
    pltpu.make_async_copy(stg_b, out_unit(0, 0, 0), sem_b).wait()


def kernel(indices, table):
    idx_t = indices.T
    table_flat = table.reshape(-1)
    out = _sc_gather(idx_t, table_flat)
    return out.transpose(2, 0, 1)


# speedup vs baseline: 1.0785x; 1.0006x over previous
import functools
import jax
import jax.numpy as jnp
from jax import lax
from jax.experimental import pallas as pl
from jax.experimental.pallas import tpu as pltpu
from jax.experimental.pallas import tpu_sc as plsc

_NUM_SETS = 26
_SET_LEN = 10
_EMBED_DIM = 64
_BATCH = 16384
_NUM_ROWS = _NUM_SETS * _SET_LEN

_L = 16
_NC, _NS = 2, 16
_NW = _NC * _NS
_CT = _EMBED_DIM // 8
_BCH = 4
_TPU_B = _BATCH // (_BCH * 128)
_UNITS = _NUM_SETS * _CT * _BCH
_U_PER_W = _UNITS // _NW

_mesh = plsc.VectorSubcoreMesh(core_axis_name="c", subcore_axis_name="s")


@functools.partial(
    pl.kernel,
    out_type=jax.ShapeDtypeStruct((_NUM_SETS, _EMBED_DIM, _BATCH), jnp.float32),
    mesh=_mesh,
    compiler_params=pltpu.CompilerParams(
        needs_layout_passes=False,
        use_tc_tiling_on_sc=True,
        disable_bounds_checks=True,
    ),
    scratch_types=[
        pltpu.VMEM((_NUM_ROWS * _EMBED_DIM,), jnp.float32),
        pltpu.VMEM((2, _BATCH), jnp.int32),
        pltpu.VMEM((1, 8, _TPU_B * 128), jnp.float32),
        pltpu.VMEM((1, 8, _TPU_B * 128), jnp.float32),
        pltpu.SemaphoreType.DMA,
        pltpu.SemaphoreType.DMA,
    ],
)
def _sc_gather(idx_hbm, table_hbm, out_hbm, table_v, idx_v, stg_a, stg_b,
               sem_a, sem_b):
    wid = lax.axis_index("s") * _NC + lax.axis_index("c")
    ubase = wid * _U_PER_W
    s0 = ubase // (_CT * _BCH)

    s_last = (ubase + _U_PER_W - 1) // (_CT * _BCH)
    tbl_cp = pltpu.async_copy(table_hbm, table_v, sem_a)
    i0_cp = pltpu.async_copy(idx_hbm.at[pl.ds(s0, 1)], idx_v.at[pl.ds(0, 1)],
                             sem_b)
    i1_cp = pltpu.async_copy(idx_hbm.at[pl.ds(s_last, 1)],
                             idx_v.at[pl.ds(1, 1)], sem_b)
    tbl_cp.wait()
    i0_cp.wait()
    i1_cp.wait()

    lane = lax.iota(jnp.int32, _L)
    lane_c = jnp.minimum(lane, _SET_LEN - 1)

    def out_unit(s, ct, ch):
        return out_hbm.at[
            pl.ds(s, 1), pl.ds(ct * 8, 8), pl.ds(ch * (_TPU_B * 128), _TPU_B * 128)
        ]

    def do_unit(ul, stg, sem, drain):
        u = ubase + ul
        s = u // (_CT * _BCH)
        r = lax.rem(u, _CT * _BCH)
        ct = r // _BCH
        ch = lax.rem(r, _BCH)
        irow = jnp.where(s == s0, 0, 1)
        if drain:
            pltpu.make_async_copy(stg, out_unit(s, ct, ch), sem).wait()
        tvecs = [
            plsc.load_gather(
                table_v,
                [s * (_SET_LEN * _EMBED_DIM) + lane_c * _EMBED_DIM
                 + (ct * 8 + i)],
            )
            for i in range(8)
        ]

        @pl.loop(0, _TPU_B)
        def _(t):
            b0 = ch * (_TPU_B * 128) + t * 128
            ivs = [idx_v[irow, pl.ds(b0 + g * _L, _L)] for g in range(8)]
            for i in range(8):
                for g in range(8):
                    stg[0, i, pl.ds(t * 128 + g * _L, _L)] = jnp.take_along_axis(
                        tvecs[i], ivs[g], axis=0, mode="promise_in_bounds"
                    )

        pltpu.async_copy(stg, out_unit(s, ct, ch), sem)

    do_unit(0, stg_a, sem_a, False)
    do_unit(1, stg_b, sem_b, False)

    @pl.loop(2, _U_PER_W, step=2)
    def _(ul):
        do_unit(ul, stg_a, sem_a, True)
        do_unit(ul + 1, stg_b, sem_b, True)

    pltpu.make_async_copy(stg_a, out_unit(0, 0, 0), sem_a).wait()
    pltpu.make_async_copy(stg_b, out_unit(0, 0, 0), sem_b).wait()


def kernel(indices, table):
    idx_t = indices.T
    table_flat = table.reshape(-1)
    out = _sc_gather(idx_t, table_flat)
    return out.transpose(2, 0, 1)
